# baseline (speedup 1.0000x reference)
import jax
import jax.numpy as jnp
from jax import lax
from jax.experimental import pallas as pl
from jax.experimental.pallas import tpu as pltpu
from jax.experimental.pallas import tpu_sc as plsc

N = 16384
K = 128
NC = 2
NS = 16
NW = NC * NS
L = 16
ROWS_PER_W = N // NW
CHUNK = 64
NCHUNK = ROWS_PER_W // CHUNK
NBUF = 2


def _neg_sqrt16(x):
    x = jnp.maximum(x, jnp.float32(1e-30))
    i = lax.bitcast_convert_type(x, jnp.int32)
    i = jnp.int32(0x5F3759DF) - lax.shift_right_arithmetic(i, 1)
    y = lax.bitcast_convert_type(i, jnp.float32)
    h = x * jnp.float32(0.5)
    for _ in range(3):
        y = y * (jnp.float32(1.5) - h * y * y)
    return -(x * y)


def _score_body(x_hbm, out_hbm, buf0, buf1, res, sem0, sem1):
    wid = lax.axis_index("s") * NC + lax.axis_index("c")
    base = wid * ROWS_PER_W
    bufs = (buf0, buf1)
    sems = (sem0, sem1)

    def start(c, b):
        row0 = base + c * CHUNK
        pltpu.async_copy(
            x_hbm.at[pl.ds(row0, CHUNK)],
            bufs[b].reshape(CHUNK, 3, K), sems[b]
        )

    start(0, 0)
    start(1, 1)

    iota = lax.iota(jnp.int32, L)

    @pl.loop(0, NCHUNK, step=NBUF)
    def _outer(c0):
        for b in range(NBUF):
            c = c0 + b
            buf = bufs[b]
            pltpu.make_async_copy(
                x_hbm.at[pl.ds(0, CHUNK)],
                buf.reshape(CHUNK, 3, K), sems[b]
            ).wait()
            for g in range(CHUNK // L):
                rs = (iota + jnp.int32(g * L)) * jnp.int32(3)
                rp = rs + jnp.int32(1)
                ro = rs + jnp.int32(2)
                acc0 = jnp.zeros((L,), jnp.float32)

                @pl.loop(0, K, init_carry=(acc0, iota), unroll=8)
                def _k(_, carry):
                    acc, col = carry
                    s = plsc.load_gather(buf, [rs, col])
                    p = plsc.load_gather(buf, [rp, col])
                    o = plsc.load_gather(buf, [ro, col])
                    d = (s + p) - o
                    return acc + d * d, (col + jnp.int32(1)) & jnp.int32(K - 1)

                acc, _ = _k
                res[pl.ds(c * CHUNK + g * L, L)] = _neg_sqrt16(acc)

            @pl.when(c + NBUF < NCHUNK)
            def _():
                start(c + NBUF, b)

    pltpu.sync_copy(res, out_hbm.at[pl.ds(base, ROWS_PER_W)])


@jax.jit
def _score(x):
    mesh = plsc.VectorSubcoreMesh(core_axis_name="c", subcore_axis_name="s")
    return pl.kernel(
        _score_body,
        out_type=jax.ShapeDtypeStruct((N,), jnp.float32),
        mesh=mesh,
        compiler_params=pltpu.CompilerParams(
            needs_layout_passes=False,
            skip_device_barrier=True,
            disable_bounds_checks=True,
            disable_semaphore_checks=True,
            use_tc_tiling_on_sc=True,
        ),
        scratch_types=[
            pltpu.VMEM((CHUNK * 3, K), jnp.float32),
            pltpu.VMEM((CHUNK * 3, K), jnp.float32),
            pltpu.VMEM((ROWS_PER_W,), jnp.float32),
            pltpu.SemaphoreType.DMA,
            pltpu.SemaphoreType.DMA,
        ],
    )(x)


def _tc_whole_body(x_ref, o_ref):
    x = x_ref[...]
    w = jnp.where(lax.broadcasted_iota(jnp.int32, (1, 3, 1), 1) == 2,
                  jnp.float32(-1.0), jnp.float32(1.0))
    d = jnp.sum(x * w, axis=1)
    o_ref[...] = -jnp.sqrt(jnp.sum(d * d, axis=-1))


@jax.jit
def _score_tc_whole(x):
    return pl.pallas_call(
        _tc_whole_body,
        out_shape=jax.ShapeDtypeStruct((N,), jnp.float32),
        in_specs=[pl.BlockSpec((N, 3, K), lambda: (0, 0, 0))],
        out_specs=pl.BlockSpec((N,), lambda: (0,)),
        compiler_params=pltpu.CompilerParams(
            vmem_limit_bytes=120 * 1024 * 1024,
        ),
    )(x)


def kernel(triples):
    return _score_tc_whole(triples)


# speedup vs baseline: 1.1291x; 1.1291x over previous
import jax
import jax.numpy as jnp
from jax import lax
from jax.experimental import pallas as pl
from jax.experimental.pallas import tpu as pltpu
from jax.experimental.pallas import tpu_sc as plsc

N = 16384
K = 128
NC = 2
NS = 16
NW = NC * NS
L = 16
ROWS_PER_W = N // NW
CHUNK = 128
NCHUNK = ROWS_PER_W // CHUNK
NBUF = 2


def _neg_sqrt16(x):
    x = jnp.maximum(x, jnp.float32(1e-30))
    i = lax.bitcast_convert_type(x, jnp.int32)
    i = jnp.int32(0x5F3759DF) - lax.shift_right_arithmetic(i, 1)
    y = lax.bitcast_convert_type(i, jnp.float32)
    h = x * jnp.float32(0.5)
    for _ in range(3):
        y = y * (jnp.float32(1.5) - h * y * y)
    return -(x * y)


def _score_body(x_hbm, out_hbm, buf0, buf1, res, sem0, sem1):
    wid = lax.axis_index("s") * NC + lax.axis_index("c")
    base = wid * ROWS_PER_W
    bufs = (buf0, buf1)
    sems = (sem0, sem1)

    def start(c, b):
        row0 = base + c * CHUNK
        pltpu.async_copy(
            x_hbm.at[pl.ds(row0, CHUNK)],
            bufs[b].reshape(CHUNK, 3, K), sems[b]
        )

    start(0, 0)
    start(1, 1)

    iota = lax.iota(jnp.int32, L)

    @pl.loop(0, NCHUNK, step=NBUF)
    def _outer(c0):
        for b in range(NBUF):
            c = c0 + b
            buf = bufs[b]
            pltpu.make_async_copy(
                x_hbm.at[pl.ds(0, CHUNK)],
                buf.reshape(CHUNK, 3, K), sems[b]
            ).wait()
            for g in range(CHUNK // L):
                rs = (iota + jnp.int32(g * L)) * jnp.int32(3)
                rp = rs + jnp.int32(1)
                ro = rs + jnp.int32(2)
                acc0 = jnp.zeros((L,), jnp.float32)

                @pl.loop(0, K, init_carry=(acc0, iota), unroll=16)
                def _k(_, carry):
                    acc, col = carry
                    s = plsc.load_gather(buf, [rs, col])
                    p = plsc.load_gather(buf, [rp, col])
                    o = plsc.load_gather(buf, [ro, col])
                    d = (s + p) - o
                    return acc + d * d, (col + jnp.int32(1)) & jnp.int32(K - 1)

                acc, _ = _k
                res[pl.ds(c * CHUNK + g * L, L)] = _neg_sqrt16(acc)

            @pl.when(c + NBUF < NCHUNK)
            def _():
                start(c + NBUF, b)

    pltpu.sync_copy(res, out_hbm.at[pl.ds(base, ROWS_PER_W)])


@jax.jit
def _score(x):
    mesh = plsc.VectorSubcoreMesh(core_axis_name="c", subcore_axis_name="s")
    return pl.kernel(
        _score_body,
        out_type=jax.ShapeDtypeStruct((N,), jnp.float32),
        mesh=mesh,
        compiler_params=pltpu.CompilerParams(
            needs_layout_passes=False,
            skip_device_barrier=True,
            disable_bounds_checks=True,
            disable_semaphore_checks=True,
            use_tc_tiling_on_sc=True,
        ),
        scratch_types=[
            pltpu.VMEM((CHUNK * 3, K), jnp.float32),
            pltpu.VMEM((CHUNK * 3, K), jnp.float32),
            pltpu.VMEM((ROWS_PER_W,), jnp.float32),
            pltpu.SemaphoreType.DMA,
            pltpu.SemaphoreType.DMA,
        ],
    )(x)


def kernel(triples):
    return _score(triples)


# speedup vs baseline: 1.1663x; 1.0329x over previous
import jax
import jax.numpy as jnp
from jax import lax
from jax.experimental import pallas as pl
from jax.experimental.pallas import tpu as pltpu
from jax.experimental.pallas import tpu_sc as plsc

N = 16384
K = 128
NC = 2
NS = 16
NW = NC * NS
L = 16
ROWS_PER_W = N // NW
CHUNK = 64
NCHUNK = ROWS_PER_W // CHUNK
NBUF = 2


def _neg_sqrt16(x):
    x = jnp.maximum(x, jnp.float32(1e-30))
    i = lax.bitcast_convert_type(x, jnp.int32)
    i = jnp.int32(0x5F3759DF) - lax.shift_right_arithmetic(i, 1)
    y = lax.bitcast_convert_type(i, jnp.float32)
    h = x * jnp.float32(0.5)
    for _ in range(3):
        y = y * (jnp.float32(1.5) - h * y * y)
    return -(x * y)


def _score_body(x_hbm, out_hbm, buf0, buf1, res, sem0, sem1):
    wid = lax.axis_index("s") * NC + lax.axis_index("c")
    base = wid * ROWS_PER_W
    bufs = (buf0, buf1)
    sems = (sem0, sem1)

    def start(c, b):
        row0 = base + c * CHUNK
        pltpu.async_copy(
            x_hbm.at[pl.ds(row0, CHUNK)],
            bufs[b].reshape(CHUNK, 3, K), sems[b]
        )

    start(0, 0)
    start(1, 1)

    iota = lax.iota(jnp.int32, L)

    @pl.loop(0, NCHUNK, step=NBUF)
    def _outer(c0):
        for b in range(NBUF):
            c = c0 + b
            buf = bufs[b]
            pltpu.make_async_copy(
                x_hbm.at[pl.ds(0, CHUNK)],
                buf.reshape(CHUNK, 3, K), sems[b]
            ).wait()
            for g in range(CHUNK // L):
                rs = (iota + jnp.int32(g * L)) * jnp.int32(3)
                rp = rs + jnp.int32(1)
                ro = rs + jnp.int32(2)
                acc0 = jnp.zeros((L,), jnp.float32)

                @pl.loop(0, K, init_carry=(acc0, iota), unroll=16)
                def _k(_, carry):
                    acc, col = carry
                    s = plsc.load_gather(buf, [rs, col])
                    p = plsc.load_gather(buf, [rp, col])
                    o = plsc.load_gather(buf, [ro, col])
                    d = (s + p) - o
                    return acc + d * d, (col + jnp.int32(1)) & jnp.int32(K - 1)

                acc, _ = _k
                res[pl.ds(c * CHUNK + g * L, L)] = _neg_sqrt16(acc)

            @pl.when(c + NBUF < NCHUNK)
            def _():
                start(c + NBUF, b)

    pltpu.sync_copy(res, out_hbm.at[pl.ds(base, ROWS_PER_W)])


@jax.jit
def _score(x):
    mesh = plsc.VectorSubcoreMesh(core_axis_name="c", subcore_axis_name="s")
    return pl.kernel(
        _score_body,
        out_type=jax.ShapeDtypeStruct((N,), jnp.float32),
        mesh=mesh,
        compiler_params=pltpu.CompilerParams(
            needs_layout_passes=False,
            skip_device_barrier=True,
            disable_bounds_checks=True,
            disable_semaphore_checks=True,
            use_tc_tiling_on_sc=True,
        ),
        scratch_types=[
            pltpu.VMEM((CHUNK * 3, K), jnp.float32),
            pltpu.VMEM((CHUNK * 3, K), jnp.float32),
            pltpu.VMEM((ROWS_PER_W,), jnp.float32),
            pltpu.SemaphoreType.DMA,
            pltpu.SemaphoreType.DMA,
        ],
    )(x)


def kernel(triples):
    return _score(triples)


# speedup vs baseline: 1.6099x; 1.3803x over previous
import jax
import jax.numpy as jnp
from jax import lax
from jax.experimental import pallas as pl
from jax.experimental.pallas import tpu as pltpu
from jax.experimental.pallas import tpu_sc as plsc

N = 16384
K = 128
NC = 2
NS = 16
NW = NC * NS
L = 16
ROWS_PER_W = N // NW
CHUNK = 64
NCHUNK = ROWS_PER_W // CHUNK
NBUF = 2


def _neg_sqrt16(x):
    x = jnp.maximum(x, jnp.float32(1e-30))
    i = lax.bitcast_convert_type(x, jnp.int32)
    i = jnp.int32(0x5F3759DF) - lax.shift_right_arithmetic(i, 1)
    y = lax.bitcast_convert_type(i, jnp.float32)
    h = x * jnp.float32(0.5)
    for _ in range(3):
        y = y * (jnp.float32(1.5) - h * y * y)
    return -(x * y)


def _score_body(s_hbm, p_hbm, o_hbm, out_hbm,
                sbuf0, pbuf0, obuf0, sbuf1, pbuf1, obuf1, res, sem0, sem1):
    wid = lax.axis_index("s") * NC + lax.axis_index("c")
    base = wid * ROWS_PER_W
    bufs = ((sbuf0, pbuf0, obuf0), (sbuf1, pbuf1, obuf1))
    srcs = (s_hbm, p_hbm, o_hbm)
    sems = (sem0, sem1)

    def start(c, b):
        row0 = base + c * CHUNK
        for w in range(3):
            pltpu.async_copy(
                srcs[w].at[pl.ds(row0, CHUNK)], bufs[b][w], sems[b]
            )

    start(0, 0)
    start(1, 1)

    iota = lax.iota(jnp.int32, L)

    @pl.loop(0, NCHUNK, step=NBUF)
    def _outer(c0):
        for b in range(NBUF):
            c = c0 + b
            sb, pb, ob = bufs[b]
            for w in range(3):
                pltpu.make_async_copy(
                    srcs[w].at[pl.ds(0, CHUNK)], bufs[b][w], sems[b]
                ).wait()
            for g in range(CHUNK // L):
                rows = iota + jnp.int32(g * L)
                acc0 = jnp.zeros((L,), jnp.float32)

                @pl.loop(0, K, init_carry=(acc0, iota), unroll=8)
                def _k(_, carry):
                    acc, col = carry
                    s = plsc.load_gather(sb, [rows, col])
                    p = plsc.load_gather(pb, [rows, col])
                    o = plsc.load_gather(ob, [rows, col])
                    d = (s + p) - o
                    return acc + d * d, (col + jnp.int32(1)) & jnp.int32(K - 1)

                acc, _ = _k
                res[pl.ds(c * CHUNK + g * L, L)] = _neg_sqrt16(acc)

            @pl.when(c + NBUF < NCHUNK)
            def _():
                start(c + NBUF, b)

    pltpu.sync_copy(res, out_hbm.at[pl.ds(base, ROWS_PER_W)])


@jax.jit
def _score(x):
    mesh = plsc.VectorSubcoreMesh(core_axis_name="c", subcore_axis_name="s")
    return pl.kernel(
        _score_body,
        out_type=jax.ShapeDtypeStruct((N,), jnp.float32),
        mesh=mesh,
        compiler_params=pltpu.CompilerParams(
            needs_layout_passes=False,
            skip_device_barrier=True,
            disable_bounds_checks=True,
            disable_semaphore_checks=True,
            use_tc_tiling_on_sc=True,
        ),
        scratch_types=[
            pltpu.VMEM((CHUNK, K), jnp.float32),
            pltpu.VMEM((CHUNK, K), jnp.float32),
            pltpu.VMEM((CHUNK, K), jnp.float32),
            pltpu.VMEM((CHUNK, K), jnp.float32),
            pltpu.VMEM((CHUNK, K), jnp.float32),
            pltpu.VMEM((CHUNK, K), jnp.float32),
            pltpu.VMEM((ROWS_PER_W,), jnp.float32),
            pltpu.SemaphoreType.DMA,
            pltpu.SemaphoreType.DMA,
        ],
    )(x[:, 0, :], x[:, 1, :], x[:, 2, :])


def kernel(triples):
    return _score(triples)


# speedup vs baseline: 1.6261x; 1.0101x over previous
import jax
import jax.numpy as jnp
from jax import lax
from jax.experimental import pallas as pl
from jax.experimental.pallas import tpu as pltpu
from jax.experimental.pallas import tpu_sc as plsc

N = 16384
K = 128
NC = 2
NS = 16
NW = NC * NS
L = 16
ROWS_PER_W = N // NW
CHUNK = 64
NCHUNK = ROWS_PER_W // CHUNK
NBUF = 2


def _neg_sqrt16(x):
    x = jnp.maximum(x, jnp.float32(1e-30))
    i = lax.bitcast_convert_type(x, jnp.int32)
    i = jnp.int32(0x5F3759DF) - lax.shift_right_arithmetic(i, 1)
    y = lax.bitcast_convert_type(i, jnp.float32)
    h = x * jnp.float32(0.5)
    for _ in range(3):
        y = y * (jnp.float32(1.5) - h * y * y)
    return -(x * y)


def _score_body(s_hbm, p_hbm, o_hbm, out_hbm,
                sbuf0, pbuf0, obuf0, sbuf1, pbuf1, obuf1, res, sem0, sem1):
    wid = lax.axis_index("s") * NC + lax.axis_index("c")
    base = wid * ROWS_PER_W
    bufs = ((sbuf0, pbuf0, obuf0), (sbuf1, pbuf1, obuf1))
    srcs = (s_hbm, p_hbm, o_hbm)
    sems = (sem0, sem1)

    def start(c, b):
        row0 = base + c * CHUNK
        for w in range(3):
            pltpu.async_copy(
                srcs[w].at[pl.ds(row0, CHUNK)], bufs[b][w], sems[b]
            )

    start(0, 0)
    start(1, 1)

    iota = lax.iota(jnp.int32, L)

    @pl.loop(0, NCHUNK, step=NBUF)
    def _outer(c0):
        for b in range(NBUF):
            c = c0 + b
            sb, pb, ob = bufs[b]
            for w in range(3):
                pltpu.make_async_copy(
                    srcs[w].at[pl.ds(0, CHUNK)], bufs[b][w], sems[b]
                ).wait()
            for g in range(CHUNK // L):
                rows = iota + jnp.int32(g * L)
                acc0 = jnp.zeros((L,), jnp.float32)

                @pl.loop(0, K, init_carry=(acc0, iota), unroll=4)
                def _k(_, carry):
                    acc, col = carry
                    s = plsc.load_gather(sb, [rows, col])
                    p = plsc.load_gather(pb, [rows, col])
                    o = plsc.load_gather(ob, [rows, col])
                    d = (s + p) - o
                    return acc + d * d, (col + jnp.int32(1)) & jnp.int32(K - 1)

                acc, _ = _k
                res[pl.ds(c * CHUNK + g * L, L)] = _neg_sqrt16(acc)

            @pl.when(c + NBUF < NCHUNK)
            def _():
                start(c + NBUF, b)

    pltpu.sync_copy(res, out_hbm.at[pl.ds(base, ROWS_PER_W)])


@jax.jit
def _score(x):
    mesh = plsc.VectorSubcoreMesh(core_axis_name="c", subcore_axis_name="s")
    return pl.kernel(
        _score_body,
        out_type=jax.ShapeDtypeStruct((N,), jnp.float32),
        mesh=mesh,
        compiler_params=pltpu.CompilerParams(
            needs_layout_passes=False,
            skip_device_barrier=True,
            disable_bounds_checks=True,
            disable_semaphore_checks=True,
            use_tc_tiling_on_sc=True,
        ),
        scratch_types=[
            pltpu.VMEM((CHUNK, K), jnp.float32),
            pltpu.VMEM((CHUNK, K), jnp.float32),
            pltpu.VMEM((CHUNK, K), jnp.float32),
            pltpu.VMEM((CHUNK, K), jnp.float32),
            pltpu.VMEM((CHUNK, K), jnp.float32),
            pltpu.VMEM((CHUNK, K), jnp.float32),
            pltpu.VMEM((ROWS_PER_W,), jnp.float32),
            pltpu.SemaphoreType.DMA,
            pltpu.SemaphoreType.DMA,
        ],
    )(x[:, 0, :], x[:, 1, :], x[:, 2, :])


def kernel(triples):
    return _score(triples)


# speedup vs baseline: 2.1407x; 1.3165x over previous
import jax
import jax.numpy as jnp
from jax import lax
from jax.experimental import pallas as pl
from jax.experimental.pallas import tpu as pltpu
from jax.experimental.pallas import tpu_sc as plsc

N = 16384
K = 128
NC = 2
NS = 16
NW = NC * NS
L = 16
ROWS_PER_W = N // NW
CHUNK = 64
NCHUNK = ROWS_PER_W // CHUNK
NBUF = 2


def _neg_sqrt16(x):
    x = jnp.maximum(x, jnp.float32(1e-30))
    i = lax.bitcast_convert_type(x, jnp.int32)
    i = jnp.int32(0x5F3759DF) - lax.shift_right_arithmetic(i, 1)
    y = lax.bitcast_convert_type(i, jnp.float32)
    h = x * jnp.float32(0.5)
    for _ in range(3):
        y = y * (jnp.float32(1.5) - h * y * y)
    return -(x * y)


def _score_body(s_hbm, p_hbm, o_hbm, out_hbm,
                sbuf0, pbuf0, obuf0, sbuf1, pbuf1, obuf1, res, sem0, sem1):
    wid = lax.axis_index("s") * NC + lax.axis_index("c")
    base = wid * ROWS_PER_W
    bufs = ((sbuf0, pbuf0, obuf0), (sbuf1, pbuf1, obuf1))
    srcs = (s_hbm, p_hbm, o_hbm)
    sems = (sem0, sem1)

    def start(c, b):
        row0 = base + c * CHUNK
        for w in range(3):
            pltpu.async_copy(
                srcs[w].at[pl.ds(row0, CHUNK)], bufs[b][w], sems[b]
            )

    start(0, 0)
    start(1, 1)

    iota = lax.iota(jnp.int32, L)

    @pl.loop(0, NCHUNK, step=NBUF)
    def _outer(c0):
        for b in range(NBUF):
            c = c0 + b
            sb, pb, ob = bufs[b]
            for w in range(3):
                pltpu.make_async_copy(
                    srcs[w].at[pl.ds(0, CHUNK)], bufs[b][w], sems[b]
                ).wait()
            for g in range(CHUNK // L):
                rows = iota + jnp.int32(g * L)
                acc0 = jnp.zeros((L,), jnp.float32)

                @pl.loop(0, K, init_carry=(acc0, iota), unroll=4)
                def _k(_, carry):
                    acc, col = carry
                    s = plsc.load_gather(sb, [rows, col])
                    p = plsc.load_gather(pb, [rows, col])
                    o = plsc.load_gather(ob, [rows, col])
                    d = (s + p) - o
                    return acc + d * d, (col + jnp.int32(1)) & jnp.int32(K - 1)

                acc, _ = _k
                res[pl.ds(c * CHUNK + g * L, L)] = _neg_sqrt16(acc)

            @pl.when(c + NBUF < NCHUNK)
            def _():
                start(c + NBUF, b)

    pltpu.sync_copy(res, out_hbm.at[pl.ds(base, ROWS_PER_W)])


@jax.jit
def _score(x):
    mesh = plsc.VectorSubcoreMesh(core_axis_name="c", subcore_axis_name="s")
    return pl.kernel(
        _score_body,
        out_type=jax.ShapeDtypeStruct((N,), jnp.float32),
        mesh=mesh,
        compiler_params=pltpu.CompilerParams(
            needs_layout_passes=False,
            skip_device_barrier=True,
            disable_bounds_checks=True,
            disable_semaphore_checks=True,
            use_tc_tiling_on_sc=True,
        ),
        scratch_types=[
            pltpu.VMEM((CHUNK, K), jnp.float32),
            pltpu.VMEM((CHUNK, K), jnp.float32),
            pltpu.VMEM((CHUNK, K), jnp.float32),
            pltpu.VMEM((CHUNK, K), jnp.float32),
            pltpu.VMEM((CHUNK, K), jnp.float32),
            pltpu.VMEM((CHUNK, K), jnp.float32),
            pltpu.VMEM((ROWS_PER_W,), jnp.float32),
            pltpu.SemaphoreType.DMA,
            pltpu.SemaphoreType.DMA,
        ],
    )(x[:, 0, :], x[:, 1, :], x[:, 2, :])


_TC_BN = 1024


def _tc_body2(s_ref, p_ref, o_ref, out_ref):
    d = s_ref[...] + p_ref[...] - o_ref[...]
    out_ref[...] = -jnp.sqrt(jnp.sum(d * d, axis=-1))


@jax.jit
def _score_tc2(x):
    spec = pl.BlockSpec((_TC_BN, K), lambda i: (i, 0))
    return pl.pallas_call(
        _tc_body2,
        out_shape=jax.ShapeDtypeStruct((N,), jnp.float32),
        grid=(N // _TC_BN,),
        in_specs=[spec, spec, spec],
        out_specs=pl.BlockSpec((_TC_BN,), lambda i: (i,)),
    )(x[:, 0, :], x[:, 1, :], x[:, 2, :])


def kernel(triples):
    return _score_tc2(triples)
